# baseline (speedup 1.0000x reference)
import jax
import jax.numpy as jnp
from jax import lax
from jax.experimental import pallas as pl
from jax.experimental.pallas import tpu as pltpu
from jax.experimental.pallas import tpu_sc as plsc

N = 10000
D = 128
E = 320000
NC = 2
NS = 16
NW = NC * NS
EPW = E // NW
K = 80
CH = 128
EPP = CH * K
NP = 10112
RPT = NP // NS
G = 16
NSUP = CH // G

_mesh = plsc.VectorSubcoreMesh(
    core_axis_name="c", subcore_axis_name="s", num_cores=NC, num_subcores=NS)


def _tile_slices():
    out = []
    left, ofs = RPT, 0
    while left > 0:
        sz = K if left >= K else left
        out.append((ofs, sz))
        ofs += sz
        left -= sz
    return out


def _sc_agg_body(h_hbm, src_hbm, dst_hbm, p_out,
                 src_v, dst_sl, buf0, buf1, buf2, acc_sh, gs0, gs1, gs2, dsem):
    cid = lax.axis_index("c")
    sid = lax.axis_index("s")
    wid = cid * NS + sid
    bufs = (buf0, buf1, buf2)
    sems = (gs0, gs1, gs2)

    pltpu.sync_copy(src_hbm.at[wid], src_v)
    pltpu.sync_copy(dst_hbm.at[wid, pl.ds(0, G)], dst_sl.at[0])
    pltpu.sync_copy(dst_hbm.at[wid, pl.ds(G, G)], dst_sl.at[1])

    def zrow(r, _):
        for c16 in range(D // 16):
            buf0[r, pl.ds(c16 * 16, 16)] = jnp.zeros((16,), jnp.float32)
        return 0
    lax.fori_loop(0, K, zrow, 0)
    for ofs, sz in _tile_slices():
        pltpu.sync_copy(buf0.at[pl.ds(0, sz)],
                        acc_sh.at[pl.ds(sid * RPT + ofs, sz)])

    plsc.subcore_barrier()

    def sidx(c):
        return src_v.at[pl.ds(c * K, K)]

    def didx(c):
        return dst_sl.at[lax.rem(c // G, 2), lax.rem(c, G)]

    def step(c, j):
        @pl.when(lax.rem(c, G) == 0)
        def _():
            sup = c // G + 1

            @pl.when((sup >= 2) & (sup < NSUP))
            def _():
                pltpu.async_copy(dst_hbm.at[wid, pl.ds(sup * G, G)],
                                 dst_sl.at[lax.rem(sup, 2)], dsem)

        @pl.when(lax.rem(c, G) == G - 1)
        def _():
            sup = c // G + 1

            @pl.when((sup >= 2) & (sup < NSUP))
            def _():
                pltpu.make_async_copy(dst_hbm.at[wid, pl.ds(0, G)],
                                      dst_sl.at[lax.rem(sup, 2)], dsem).wait()

        pltpu.make_async_copy(h_hbm.at[sidx(c)], bufs[j], sems[j]).wait()
        pltpu.sync_copy(bufs[j], acc_sh.at[didx(c)], add=True)

        @pl.when(c + 3 < CH)
        def _():
            pltpu.async_copy(h_hbm.at[sidx(c + 3)], bufs[j], sems[j])

    pltpu.async_copy(h_hbm.at[sidx(0)], buf0, gs0)
    pltpu.async_copy(h_hbm.at[sidx(1)], buf1, gs1)
    pltpu.async_copy(h_hbm.at[sidx(2)], buf2, gs2)

    def body(i, _):
        c = 3 * i
        step(c, 0)
        step(c + 1, 1)
        step(c + 2, 2)
        return 0
    lax.fori_loop(0, CH // 3, body, 0)
    step(CH - 2, 0)
    step(CH - 1, 1)

    plsc.subcore_barrier()

    for ofs, sz in _tile_slices():
        s = sid * RPT + ofs
        pltpu.sync_copy(acc_sh.at[pl.ds(s, sz)], buf0.at[pl.ds(0, sz)])
        pltpu.sync_copy(buf0.at[pl.ds(0, sz)], p_out.at[cid, pl.ds(s, sz)])


_agg = pl.kernel(
    _sc_agg_body,
    out_type=jax.ShapeDtypeStruct((NC, NP, D), jnp.float32),
    mesh=_mesh,
    scratch_types=[
        pltpu.VMEM((EPP,), jnp.int32),
        pltpu.VMEM((2, G, K), jnp.int32),
        pltpu.VMEM((K, D), jnp.float32),
        pltpu.VMEM((K, D), jnp.float32),
        pltpu.VMEM((K, D), jnp.float32),
        pltpu.VMEM_SHARED((NP, D), jnp.float32),
        pltpu.SemaphoreType.DMA,
        pltpu.SemaphoreType.DMA,
        pltpu.SemaphoreType.DMA,
        pltpu.SemaphoreType.DMA,
    ],
)


def _sc_cnt_body(dst_hbm, c_out, dst_v, obuf, cnt_sh, sem0):
    cid = lax.axis_index("c")
    sid = lax.axis_index("s")
    wid = cid * NS + sid

    pltpu.sync_copy(dst_hbm.at[wid], dst_v)

    def fill(val):
        def row(r, _):
            for c16 in range(D // 16):
                obuf[r, pl.ds(c16 * 16, 16)] = jnp.full((16,), val, jnp.float32)
            return 0
        lax.fori_loop(0, K, row, 0)

    fill(0.0)
    for ofs, sz in _tile_slices():
        pltpu.sync_copy(obuf.at[pl.ds(0, sz)],
                        cnt_sh.at[pl.ds(sid * RPT + ofs, sz)])
    fill(1.0)

    plsc.subcore_barrier()

    B = 8

    def body(g, _):
        for j in range(B):
            pltpu.async_copy(obuf, cnt_sh.at[dst_v.at[g * B + j]], sem0,
                             add=True)
        for j in range(B):
            pltpu.make_async_copy(obuf, cnt_sh.at[dst_v.at[g * B + j]],
                                  sem0).wait()
        return 0
    lax.fori_loop(0, CH // B, body, 0)

    plsc.subcore_barrier()

    for ofs, sz in _tile_slices():
        s = sid * RPT + ofs
        pltpu.sync_copy(cnt_sh.at[pl.ds(s, sz)], obuf.at[pl.ds(0, sz)])
        pltpu.sync_copy(obuf.at[pl.ds(0, sz)], c_out.at[cid, pl.ds(s, sz)])


_cnt = pl.kernel(
    _sc_cnt_body,
    out_type=jax.ShapeDtypeStruct((NC, NP, D), jnp.float32),
    mesh=_mesh,
    scratch_types=[
        pltpu.VMEM((CH, K), jnp.int32),
        pltpu.VMEM((K, D), jnp.float32),
        pltpu.VMEM_SHARED((NP, D), jnp.float32),
        pltpu.SemaphoreType.DMA,
    ],
)

R = 1000


def _tc_self_body(h, wr, o_ref):
    o_ref[...] = lax.dot_general(h[...], wr[...], (((1,), (1,)), ((), ())),
                                 preferred_element_type=jnp.float32)


def _mean_from_parts(p0, p1, c0, c1):
    cnt = c0[:, 0:1] + c1[:, 0:1]
    inv = 1.0 / jnp.maximum(cnt, 1.0)
    return (p0[...] + p1[...]) * inv


def _tc_combine_body(p0, p1, c0, c1, sf, wl, bl, o_ref):
    mean = _mean_from_parts(p0, p1, c0, c1)
    acc = lax.dot_general(mean, wl[...], (((1,), (1,)), ((), ())),
                          preferred_element_type=jnp.float32)
    o_ref[...] = jnp.maximum(acc + sf[...] + bl[...], 0.0)


def _tc_final_body(p0, p1, c0, c1, sf, wl, bl, wlin, blin, o_ref):
    mean = _mean_from_parts(p0, p1, c0, c1)
    acc = lax.dot_general(mean, wl[...], (((1,), (1,)), ((), ())),
                          preferred_element_type=jnp.float32)
    hrelu = jnp.maximum(acc + sf[...] + bl[...], 0.0)
    o_ref[...] = lax.dot_general(hrelu, wlin[...], (((1,), (1,)), ((), ())),
                                 preferred_element_type=jnp.float32) + blin[...]


def _row_spec():
    return pl.BlockSpec((R, D), lambda i: (i, 0))


def _w_spec():
    return pl.BlockSpec((D, D), lambda i: (0, 0))


def _b_spec():
    return pl.BlockSpec((1, D), lambda i: (0, 0))


_tc_self = pl.pallas_call(
    _tc_self_body,
    grid=(N // R,),
    in_specs=[_row_spec(), _w_spec()],
    out_specs=_row_spec(),
    out_shape=jax.ShapeDtypeStruct((N, D), jnp.float32),
)

_tc_combine = pl.pallas_call(
    _tc_combine_body,
    grid=(N // R,),
    in_specs=[_row_spec(), _row_spec(), _row_spec(), _row_spec(),
              _row_spec(), _w_spec(), _b_spec()],
    out_specs=_row_spec(),
    out_shape=jax.ShapeDtypeStruct((N, D), jnp.float32),
)

_tc_final = pl.pallas_call(
    _tc_final_body,
    grid=(N // R,),
    in_specs=[_row_spec(), _row_spec(), _row_spec(), _row_spec(),
              _row_spec(), _w_spec(), _b_spec(), _w_spec(), _b_spec()],
    out_specs=_row_spec(),
    out_shape=jax.ShapeDtypeStruct((N, D), jnp.float32),
)


def kernel(x, edge_index, W1l, b1l, W1r, W2l, b2l, W2r, W3l, b3l, W3r,
           Wlin, blin):
    pad = EPP - EPW
    src = jnp.pad(edge_index[0].reshape(NW, EPW), ((0, 0), (0, pad)))
    dst = jnp.pad(edge_index[1].reshape(NW, EPW), ((0, 0), (0, pad)),
                  constant_values=NP - 1).reshape(NW, CH, K)
    b1 = b1l.reshape(1, D)
    b2 = b2l.reshape(1, D)
    b3 = b3l.reshape(1, D)
    bl = blin.reshape(1, D)

    c = _cnt(dst)
    s1 = _tc_self(x, W1r)
    p = _agg(x, src, dst)
    h1 = _tc_combine(p[0], p[1], c[0], c[1], s1, W1l, b1)
    s2 = _tc_self(h1, W2r)
    p = _agg(h1, src, dst)
    h2 = _tc_combine(p[0], p[1], c[0], c[1], s2, W2l, b2)
    s3 = _tc_self(h2, W3r)
    p = _agg(h2, src, dst)
    return _tc_final(p[0], p[1], c[0], c[1], s3, W3l, b3, Wlin, bl)


# speedup vs baseline: 1.0014x; 1.0014x over previous
import jax
import jax.numpy as jnp
from jax import lax
from jax.experimental import pallas as pl
from jax.experimental.pallas import tpu as pltpu
from jax.experimental.pallas import tpu_sc as plsc

N = 10000
D = 128
E = 320000
NC = 2
NS = 16
NW = NC * NS
EPW = E // NW
K = 80
CH = 128
EPP = CH * K
NP = 10112
RPT = NP // NS
G = 16
NSUP = CH // G

_mesh = plsc.VectorSubcoreMesh(
    core_axis_name="c", subcore_axis_name="s", num_cores=NC, num_subcores=NS)


def _tile_slices():
    out = []
    left, ofs = RPT, 0
    while left > 0:
        sz = K if left >= K else left
        out.append((ofs, sz))
        ofs += sz
        left -= sz
    return out


def _sc_agg_body(h_hbm, src_hbm, dst_hbm, p_out,
                 src_v, dst_sl, buf0, buf1, buf2, acc_sh, gs0, gs1, gs2, dsem):
    cid = lax.axis_index("c")
    sid = lax.axis_index("s")
    wid = cid * NS + sid
    bufs = (buf0, buf1, buf2)
    sems = (gs0, gs1, gs2)

    pltpu.sync_copy(src_hbm.at[wid], src_v)
    pltpu.sync_copy(dst_hbm.at[wid, pl.ds(0, G)], dst_sl.at[0])
    pltpu.sync_copy(dst_hbm.at[wid, pl.ds(G, G)], dst_sl.at[1])

    def zrow(r, _):
        for c16 in range(D // 16):
            buf0[r, pl.ds(c16 * 16, 16)] = jnp.zeros((16,), jnp.float32)
        return 0
    lax.fori_loop(0, K, zrow, 0)
    for ofs, sz in _tile_slices():
        pltpu.sync_copy(buf0.at[pl.ds(0, sz)],
                        acc_sh.at[pl.ds(sid * RPT + ofs, sz)])

    plsc.subcore_barrier()

    def sidx(c):
        return src_v.at[pl.ds(c * K, K)]

    def didx(c):
        return dst_sl.at[lax.rem(c // G, 2), lax.rem(c, G)]

    def step(c, j):
        @pl.when(lax.rem(c, G) == 0)
        def _():
            sup = c // G + 1

            @pl.when((sup >= 2) & (sup < NSUP))
            def _():
                pltpu.async_copy(dst_hbm.at[wid, pl.ds(sup * G, G)],
                                 dst_sl.at[lax.rem(sup, 2)], dsem)

        @pl.when(lax.rem(c, G) == G - 1)
        def _():
            sup = c // G + 1

            @pl.when((sup >= 2) & (sup < NSUP))
            def _():
                pltpu.make_async_copy(dst_hbm.at[wid, pl.ds(0, G)],
                                      dst_sl.at[lax.rem(sup, 2)], dsem).wait()

        pltpu.make_async_copy(h_hbm.at[sidx(c)], bufs[j], sems[j]).wait()
        pltpu.sync_copy(bufs[j], acc_sh.at[didx(c)], add=True)

        @pl.when(c + 3 < CH)
        def _():
            pltpu.async_copy(h_hbm.at[sidx(c + 3)], bufs[j], sems[j])

    pltpu.async_copy(h_hbm.at[sidx(0)], buf0, gs0)
    pltpu.async_copy(h_hbm.at[sidx(1)], buf1, gs1)
    pltpu.async_copy(h_hbm.at[sidx(2)], buf2, gs2)

    def body(i, _):
        c = 3 * i
        step(c, 0)
        step(c + 1, 1)
        step(c + 2, 2)
        return 0
    lax.fori_loop(0, CH // 3, body, 0)
    step(CH - 2, 0)
    step(CH - 1, 1)

    plsc.subcore_barrier()

    for ofs, sz in _tile_slices():
        s = sid * RPT + ofs
        pltpu.sync_copy(acc_sh.at[pl.ds(s, sz)], buf0.at[pl.ds(0, sz)])
        pltpu.sync_copy(buf0.at[pl.ds(0, sz)], p_out.at[cid, pl.ds(s, sz)])


_agg = pl.kernel(
    _sc_agg_body,
    out_type=jax.ShapeDtypeStruct((NC, NP, D), jnp.float32),
    mesh=_mesh,
    scratch_types=[
        pltpu.VMEM((EPP,), jnp.int32),
        pltpu.VMEM((2, G, K), jnp.int32),
        pltpu.VMEM((K, D), jnp.float32),
        pltpu.VMEM((K, D), jnp.float32),
        pltpu.VMEM((K, D), jnp.float32),
        pltpu.VMEM_SHARED((NP, D), jnp.float32),
        pltpu.SemaphoreType.DMA,
        pltpu.SemaphoreType.DMA,
        pltpu.SemaphoreType.DMA,
        pltpu.SemaphoreType.DMA,
    ],
)


def _sc_cnt_body(dst_hbm, c_out, dst_v, obuf, cnt_sh, sem0):
    cid = lax.axis_index("c")
    sid = lax.axis_index("s")
    wid = cid * NS + sid

    pltpu.sync_copy(dst_hbm.at[wid], dst_v)

    def fill(val):
        def row(r, _):
            for c16 in range(D // 16):
                obuf[r, pl.ds(c16 * 16, 16)] = jnp.full((16,), val, jnp.float32)
            return 0
        lax.fori_loop(0, K, row, 0)

    fill(0.0)
    for ofs, sz in _tile_slices():
        pltpu.sync_copy(obuf.at[pl.ds(0, sz)],
                        cnt_sh.at[pl.ds(sid * RPT + ofs, sz)])
    fill(1.0)

    plsc.subcore_barrier()

    B = 8

    def body(g, _):
        for j in range(B):
            pltpu.async_copy(obuf, cnt_sh.at[dst_v.at[g * B + j]], sem0,
                             add=True)
        for j in range(B):
            pltpu.make_async_copy(obuf, cnt_sh.at[dst_v.at[g * B + j]],
                                  sem0).wait()
        return 0
    lax.fori_loop(0, CH // B, body, 0)

    plsc.subcore_barrier()

    for ofs, sz in _tile_slices():
        s = sid * RPT + ofs
        pltpu.sync_copy(cnt_sh.at[pl.ds(s, sz)], obuf.at[pl.ds(0, sz)])
        pltpu.sync_copy(obuf.at[pl.ds(0, sz)], c_out.at[cid, pl.ds(s, sz)])


_cnt = pl.kernel(
    _sc_cnt_body,
    out_type=jax.ShapeDtypeStruct((NC, NP, D), jnp.float32),
    mesh=_mesh,
    scratch_types=[
        pltpu.VMEM((CH, K), jnp.int32),
        pltpu.VMEM((K, D), jnp.float32),
        pltpu.VMEM_SHARED((NP, D), jnp.float32),
        pltpu.SemaphoreType.DMA,
    ],
)

R = 1000


def _tc_self_body(h, wr, o_ref):
    o_ref[...] = lax.dot_general(h[...], wr[...], (((1,), (1,)), ((), ())),
                                 preferred_element_type=jnp.float32)


def _mean_from_parts(p0, p1, c0, c1):
    cnt = c0[:, 0:1] + c1[:, 0:1]
    inv = 1.0 / jnp.maximum(cnt, 1.0)
    return (p0[...] + p1[...]) * inv


def _tc_combine_body(p0, p1, c0, c1, sf, wl, bl, o_ref):
    mean = _mean_from_parts(p0, p1, c0, c1)
    acc = lax.dot_general(mean, wl[...], (((1,), (1,)), ((), ())),
                          preferred_element_type=jnp.float32)
    o_ref[...] = jnp.maximum(acc + sf[...] + bl[...], 0.0)


def _tc_final_body(p0, p1, c0, c1, sf, wl, bl, wlin, blin, o_ref):
    mean = _mean_from_parts(p0, p1, c0, c1)
    acc = lax.dot_general(mean, wl[...], (((1,), (1,)), ((), ())),
                          preferred_element_type=jnp.float32)
    hrelu = jnp.maximum(acc + sf[...] + bl[...], 0.0)
    o_ref[...] = lax.dot_general(hrelu, wlin[...], (((1,), (1,)), ((), ())),
                                 preferred_element_type=jnp.float32) + blin[...]


def _row_spec():
    return pl.BlockSpec((R, D), lambda i: (i, 0))


def _w_spec():
    return pl.BlockSpec((D, D), lambda i: (0, 0))


def _b_spec():
    return pl.BlockSpec((1, D), lambda i: (0, 0))


_tc_self = pl.pallas_call(
    _tc_self_body,
    grid=(N // R,),
    in_specs=[_row_spec(), _w_spec()],
    out_specs=_row_spec(),
    out_shape=jax.ShapeDtypeStruct((N, D), jnp.float32),
)

_tc_combine = pl.pallas_call(
    _tc_combine_body,
    grid=(N // R,),
    in_specs=[_row_spec(), _row_spec(), _row_spec(), _row_spec(),
              _row_spec(), _w_spec(), _b_spec()],
    out_specs=_row_spec(),
    out_shape=jax.ShapeDtypeStruct((N, D), jnp.float32),
)

_tc_final = pl.pallas_call(
    _tc_final_body,
    grid=(N // R,),
    in_specs=[_row_spec(), _row_spec(), _row_spec(), _row_spec(),
              _row_spec(), _w_spec(), _b_spec(), _w_spec(), _b_spec()],
    out_specs=_row_spec(),
    out_shape=jax.ShapeDtypeStruct((N, D), jnp.float32),
)


def kernel(x, edge_index, W1l, b1l, W1r, W2l, b2l, W2r, W3l, b3l, W3r,
           Wlin, blin):
    pad = EPP - EPW
    src = jnp.pad(edge_index[0].reshape(NW, EPW), ((0, 0), (0, pad)))
    pad_dst = N + (jnp.arange(pad, dtype=jnp.int32) % (NP - N))
    dst = jnp.concatenate(
        [edge_index[1].reshape(NW, EPW),
         jnp.broadcast_to(pad_dst, (NW, pad))], axis=1).reshape(NW, CH, K)
    b1 = b1l.reshape(1, D)
    b2 = b2l.reshape(1, D)
    b3 = b3l.reshape(1, D)
    bl = blin.reshape(1, D)

    c = _cnt(dst)
    s1 = _tc_self(x, W1r)
    p = _agg(x, src, dst)
    h1 = _tc_combine(p[0], p[1], c[0], c[1], s1, W1l, b1)
    s2 = _tc_self(h1, W2r)
    p = _agg(h1, src, dst)
    h2 = _tc_combine(p[0], p[1], c[0], c[1], s2, W2l, b2)
    s3 = _tc_self(h2, W3r)
    p = _agg(h2, src, dst)
    return _tc_final(p[0], p[1], c[0], c[1], s3, W3l, b3, Wlin, bl)


# speedup vs baseline: 1.0020x; 1.0006x over previous
import jax
import jax.numpy as jnp
from jax import lax
from jax.experimental import pallas as pl
from jax.experimental.pallas import tpu as pltpu
from jax.experimental.pallas import tpu_sc as plsc

N = 10000
D = 128
E = 320000
NC = 2
NS = 16
NW = NC * NS
EPW = E // NW
K = 80
CH = 128
EPP = CH * K
NP = 10240
RPT = NP // NS
G = 16
NSUP = CH // G

_mesh = plsc.VectorSubcoreMesh(
    core_axis_name="c", subcore_axis_name="s", num_cores=NC, num_subcores=NS)


def _tile_slices():
    out = []
    left, ofs = RPT, 0
    while left > 0:
        sz = K if left >= K else left
        out.append((ofs, sz))
        ofs += sz
        left -= sz
    return out


def _sc_agg_body(h_hbm, src_hbm, dst_hbm, p_out,
                 src_v, dst_sl, buf0, buf1, buf2, acc_sh, gs0, gs1, gs2, dsem):
    cid = lax.axis_index("c")
    sid = lax.axis_index("s")
    wid = cid * NS + sid
    bufs = (buf0, buf1, buf2)
    sems = (gs0, gs1, gs2)

    pltpu.sync_copy(src_hbm.at[wid], src_v)
    pltpu.sync_copy(dst_hbm.at[wid, pl.ds(0, G)], dst_sl.at[0])
    pltpu.sync_copy(dst_hbm.at[wid, pl.ds(G, G)], dst_sl.at[1])

    def zrow(r, _):
        for c16 in range(D // 16):
            buf0[r, pl.ds(c16 * 16, 16)] = jnp.zeros((16,), jnp.float32)
        return 0
    lax.fori_loop(0, K, zrow, 0)
    for ofs, sz in _tile_slices():
        pltpu.sync_copy(buf0.at[pl.ds(0, sz)],
                        acc_sh.at[pl.ds(sid * RPT + ofs, sz)])

    plsc.subcore_barrier()

    def sidx(c):
        return src_v.at[pl.ds(c * K, K)]

    def didx(c):
        return dst_sl.at[lax.rem(c // G, 2), lax.rem(c, G)]

    def step(c, j):
        @pl.when(lax.rem(c, G) == 0)
        def _():
            sup = c // G + 1

            @pl.when((sup >= 2) & (sup < NSUP))
            def _():
                pltpu.async_copy(dst_hbm.at[wid, pl.ds(sup * G, G)],
                                 dst_sl.at[lax.rem(sup, 2)], dsem)

        @pl.when(lax.rem(c, G) == G - 1)
        def _():
            sup = c // G + 1

            @pl.when((sup >= 2) & (sup < NSUP))
            def _():
                pltpu.make_async_copy(dst_hbm.at[wid, pl.ds(0, G)],
                                      dst_sl.at[lax.rem(sup, 2)], dsem).wait()

        pltpu.make_async_copy(h_hbm.at[sidx(c)], bufs[j], sems[j]).wait()
        pltpu.sync_copy(bufs[j], acc_sh.at[didx(c)], add=True)

        @pl.when(c + 3 < CH)
        def _():
            pltpu.async_copy(h_hbm.at[sidx(c + 3)], bufs[j], sems[j])

    pltpu.async_copy(h_hbm.at[sidx(0)], buf0, gs0)
    pltpu.async_copy(h_hbm.at[sidx(1)], buf1, gs1)
    pltpu.async_copy(h_hbm.at[sidx(2)], buf2, gs2)

    def body(i, _):
        c = 3 * i
        step(c, 0)
        step(c + 1, 1)
        step(c + 2, 2)
        return 0
    lax.fori_loop(0, CH // 3, body, 0)
    step(CH - 2, 0)
    step(CH - 1, 1)

    plsc.subcore_barrier()

    for ofs, sz in _tile_slices():
        s = sid * RPT + ofs
        pltpu.sync_copy(acc_sh.at[pl.ds(s, sz)], buf0.at[pl.ds(0, sz)])
        pltpu.sync_copy(buf0.at[pl.ds(0, sz)], p_out.at[cid, pl.ds(s, sz)])


_agg = pl.kernel(
    _sc_agg_body,
    out_type=jax.ShapeDtypeStruct((NC, NP, D), jnp.float32),
    mesh=_mesh,
    scratch_types=[
        pltpu.VMEM((EPP,), jnp.int32),
        pltpu.VMEM((2, G, K), jnp.int32),
        pltpu.VMEM((K, D), jnp.float32),
        pltpu.VMEM((K, D), jnp.float32),
        pltpu.VMEM((K, D), jnp.float32),
        pltpu.VMEM_SHARED((NP, D), jnp.float32),
        pltpu.SemaphoreType.DMA,
        pltpu.SemaphoreType.DMA,
        pltpu.SemaphoreType.DMA,
        pltpu.SemaphoreType.DMA,
    ],
)


def _sc_cnt_body(dst_hbm, c_out, dst_v, obuf, cnt_sh, sem0):
    cid = lax.axis_index("c")
    sid = lax.axis_index("s")
    wid = cid * NS + sid

    pltpu.sync_copy(dst_hbm.at[wid], dst_v)

    def fill(val):
        def row(r, _):
            for c16 in range(D // 16):
                obuf[r, pl.ds(c16 * 16, 16)] = jnp.full((16,), val, jnp.float32)
            return 0
        lax.fori_loop(0, K, row, 0)

    fill(0.0)
    for ofs, sz in _tile_slices():
        pltpu.sync_copy(obuf.at[pl.ds(0, sz)],
                        cnt_sh.at[pl.ds(sid * RPT + ofs, sz)])
    fill(1.0)

    plsc.subcore_barrier()

    B = 8

    def body(g, _):
        for j in range(B):
            pltpu.async_copy(obuf, cnt_sh.at[dst_v.at[g * B + j]], sem0,
                             add=True)
        for j in range(B):
            pltpu.make_async_copy(obuf, cnt_sh.at[dst_v.at[g * B + j]],
                                  sem0).wait()
        return 0
    lax.fori_loop(0, CH // B, body, 0)

    plsc.subcore_barrier()

    for ofs, sz in _tile_slices():
        s = sid * RPT + ofs
        pltpu.sync_copy(cnt_sh.at[pl.ds(s, sz)], obuf.at[pl.ds(0, sz)])
        pltpu.sync_copy(obuf.at[pl.ds(0, sz)], c_out.at[cid, pl.ds(s, sz)])


_cnt = pl.kernel(
    _sc_cnt_body,
    out_type=jax.ShapeDtypeStruct((NC, NP, D), jnp.float32),
    mesh=_mesh,
    scratch_types=[
        pltpu.VMEM((CH, K), jnp.int32),
        pltpu.VMEM((K, D), jnp.float32),
        pltpu.VMEM_SHARED((NP, D), jnp.float32),
        pltpu.SemaphoreType.DMA,
    ],
)

R = 1000


def _tc_self_body(h, wr, o_ref):
    o_ref[...] = lax.dot_general(h[...], wr[...], (((1,), (1,)), ((), ())),
                                 preferred_element_type=jnp.float32)


def _mean_from_parts(p0, p1, c0, c1):
    cnt = c0[:, 0:1] + c1[:, 0:1]
    inv = 1.0 / jnp.maximum(cnt, 1.0)
    return (p0[...] + p1[...]) * inv


def _tc_combine_body(p0, p1, c0, c1, sf, wl, bl, o_ref):
    mean = _mean_from_parts(p0, p1, c0, c1)
    acc = lax.dot_general(mean, wl[...], (((1,), (1,)), ((), ())),
                          preferred_element_type=jnp.float32)
    o_ref[...] = jnp.maximum(acc + sf[...] + bl[...], 0.0)


def _tc_final_body(p0, p1, c0, c1, sf, wl, bl, wlin, blin, o_ref):
    mean = _mean_from_parts(p0, p1, c0, c1)
    acc = lax.dot_general(mean, wl[...], (((1,), (1,)), ((), ())),
                          preferred_element_type=jnp.float32)
    hrelu = jnp.maximum(acc + sf[...] + bl[...], 0.0)
    o_ref[...] = lax.dot_general(hrelu, wlin[...], (((1,), (1,)), ((), ())),
                                 preferred_element_type=jnp.float32) + blin[...]


def _row_spec():
    return pl.BlockSpec((R, D), lambda i: (i, 0))


def _w_spec():
    return pl.BlockSpec((D, D), lambda i: (0, 0))


def _b_spec():
    return pl.BlockSpec((1, D), lambda i: (0, 0))


_tc_self = pl.pallas_call(
    _tc_self_body,
    grid=(N // R,),
    in_specs=[_row_spec(), _w_spec()],
    out_specs=_row_spec(),
    out_shape=jax.ShapeDtypeStruct((N, D), jnp.float32),
)

_tc_combine = pl.pallas_call(
    _tc_combine_body,
    grid=(N // R,),
    in_specs=[_row_spec(), _row_spec(), _row_spec(), _row_spec(),
              _row_spec(), _w_spec(), _b_spec()],
    out_specs=_row_spec(),
    out_shape=jax.ShapeDtypeStruct((N, D), jnp.float32),
)

_tc_final = pl.pallas_call(
    _tc_final_body,
    grid=(N // R,),
    in_specs=[_row_spec(), _row_spec(), _row_spec(), _row_spec(),
              _row_spec(), _w_spec(), _b_spec(), _w_spec(), _b_spec()],
    out_specs=_row_spec(),
    out_shape=jax.ShapeDtypeStruct((N, D), jnp.float32),
)


def kernel(x, edge_index, W1l, b1l, W1r, W2l, b2l, W2r, W3l, b3l, W3r,
           Wlin, blin):
    pad = EPP - EPW
    src = jnp.pad(edge_index[0].reshape(NW, EPW), ((0, 0), (0, pad)))
    pad_dst = N + (jnp.arange(pad, dtype=jnp.int32) % (NP - N))
    dst = jnp.concatenate(
        [edge_index[1].reshape(NW, EPW),
         jnp.broadcast_to(pad_dst, (NW, pad))], axis=1).reshape(NW, CH, K)
    b1 = b1l.reshape(1, D)
    b2 = b2l.reshape(1, D)
    b3 = b3l.reshape(1, D)
    bl = blin.reshape(1, D)

    c = _cnt(dst)
    s1 = _tc_self(x, W1r)
    p = _agg(x, src, dst)
    h1 = _tc_combine(p[0], p[1], c[0], c[1], s1, W1l, b1)
    s2 = _tc_self(h1, W2r)
    p = _agg(h1, src, dst)
    h2 = _tc_combine(p[0], p[1], c[0], c[1], s2, W2l, b2)
    s3 = _tc_self(h2, W3r)
    p = _agg(h2, src, dst)
    return _tc_final(p[0], p[1], c[0], c[1], s3, W3l, b3, Wlin, bl)


# speedup vs baseline: 2.0911x; 2.0869x over previous
import jax
import jax.numpy as jnp
from jax import lax
from jax.experimental import pallas as pl
from jax.experimental.pallas import tpu as pltpu
from jax.experimental.pallas import tpu_sc as plsc

N = 10000
D = 128
E = 320000
NC = 2
NS = 16
NW = NC * NS
EPW = E // NW
K = 80
CH = EPW // K
NP = 10240
RPT = NP // NS
NT = RPT // K

_mesh = plsc.VectorSubcoreMesh(
    core_axis_name="c", subcore_axis_name="s", num_cores=NC, num_subcores=NS)


def _sc_agg_body(h_hbm, src_hbm, dst_hbm, p_out,
                 src_v, dst_v, buf0, buf1, acc_sh, gs0, gs1):
    cid = lax.axis_index("c")
    sid = lax.axis_index("s")
    wid = cid * NS + sid

    pltpu.sync_copy(src_hbm.at[wid], src_v)
    pltpu.sync_copy(dst_hbm.at[wid], dst_v)

    def zrow(r, _):
        for c16 in range(D // 16):
            buf0[r, pl.ds(c16 * 16, 16)] = jnp.zeros((16,), jnp.float32)
        return 0
    lax.fori_loop(0, K, zrow, 0)
    for t in range(NT):
        pltpu.sync_copy(buf0, acc_sh.at[pl.ds(sid * RPT + t * K, K)])

    plsc.subcore_barrier()

    def sidx(c):
        return src_v.at[pl.ds(c * K, K)]

    pltpu.async_copy(h_hbm.at[sidx(0)], buf0, gs0)

    def body(o, _):
        c0 = 2 * o
        pltpu.make_async_copy(h_hbm.at[sidx(c0)], buf0, gs0).wait()
        pltpu.async_copy(h_hbm.at[sidx(c0 + 1)], buf1, gs1)
        pltpu.sync_copy(buf0, acc_sh.at[dst_v.at[c0]], add=True)
        pltpu.make_async_copy(h_hbm.at[sidx(c0 + 1)], buf1, gs1).wait()
        pltpu.async_copy(h_hbm.at[sidx(c0 + 2)], buf0, gs0)
        pltpu.sync_copy(buf1, acc_sh.at[dst_v.at[c0 + 1]], add=True)
        return 0
    lax.fori_loop(0, (CH - 1) // 2, body, 0)
    pltpu.make_async_copy(h_hbm.at[sidx(CH - 1)], buf0, gs0).wait()
    pltpu.sync_copy(buf0, acc_sh.at[dst_v.at[CH - 1]], add=True)

    plsc.subcore_barrier()

    for t in range(NT):
        s = sid * RPT + t * K
        pltpu.sync_copy(acc_sh.at[pl.ds(s, K)], buf0)
        pltpu.sync_copy(buf0, p_out.at[cid, pl.ds(s, K)])


_agg = pl.kernel(
    _sc_agg_body,
    out_type=jax.ShapeDtypeStruct((NC, NP, D), jnp.float32),
    mesh=_mesh,
    scratch_types=[
        pltpu.VMEM((EPW,), jnp.int32),
        pltpu.VMEM((CH, K), jnp.int32),
        pltpu.VMEM((K, D), jnp.float32),
        pltpu.VMEM((K, D), jnp.float32),
        pltpu.VMEM_SHARED((NP, D), jnp.float32),
        pltpu.SemaphoreType.DMA,
        pltpu.SemaphoreType.DMA,
    ],
)


def _sc_cnt_body(dst_hbm, c_out, dst_v, obuf, cnt_sh, sem0):
    cid = lax.axis_index("c")
    sid = lax.axis_index("s")
    wid = cid * NS + sid

    pltpu.sync_copy(dst_hbm.at[wid], dst_v)

    def fill(val):
        def row(r, _):
            for c16 in range(D // 16):
                obuf[r, pl.ds(c16 * 16, 16)] = jnp.full((16,), val, jnp.float32)
            return 0
        lax.fori_loop(0, K, row, 0)

    fill(0.0)
    for t in range(NT):
        pltpu.sync_copy(obuf, cnt_sh.at[pl.ds(sid * RPT + t * K, K)])
    fill(1.0)

    plsc.subcore_barrier()

    B = 5

    def body(g, _):
        for j in range(B):
            pltpu.async_copy(obuf, cnt_sh.at[dst_v.at[g * B + j]], sem0,
                             add=True)
        for j in range(B):
            pltpu.make_async_copy(obuf, cnt_sh.at[dst_v.at[g * B + j]],
                                  sem0).wait()
        return 0
    lax.fori_loop(0, CH // B, body, 0)

    plsc.subcore_barrier()

    for t in range(NT):
        s = sid * RPT + t * K
        pltpu.sync_copy(cnt_sh.at[pl.ds(s, K)], obuf)
        pltpu.sync_copy(obuf, c_out.at[cid, pl.ds(s, K)])


_cnt = pl.kernel(
    _sc_cnt_body,
    out_type=jax.ShapeDtypeStruct((NC, NP, D), jnp.float32),
    mesh=_mesh,
    scratch_types=[
        pltpu.VMEM((CH, K), jnp.int32),
        pltpu.VMEM((K, D), jnp.float32),
        pltpu.VMEM_SHARED((NP, D), jnp.float32),
        pltpu.SemaphoreType.DMA,
    ],
)

R = 1000


def _tc_self_body(h, wr, o_ref):
    o_ref[...] = lax.dot_general(h[...], wr[...], (((1,), (1,)), ((), ())),
                                 preferred_element_type=jnp.float32)


def _mean_from_parts(p0, p1, c0, c1):
    cnt = c0[:, 0:1] + c1[:, 0:1]
    inv = 1.0 / jnp.maximum(cnt, 1.0)
    return (p0[...] + p1[...]) * inv


def _tc_combine_body(p0, p1, c0, c1, sf, wl, bl, o_ref):
    mean = _mean_from_parts(p0, p1, c0, c1)
    acc = lax.dot_general(mean, wl[...], (((1,), (1,)), ((), ())),
                          preferred_element_type=jnp.float32)
    o_ref[...] = jnp.maximum(acc + sf[...] + bl[...], 0.0)


def _tc_final_body(p0, p1, c0, c1, sf, wl, bl, wlin, blin, o_ref):
    mean = _mean_from_parts(p0, p1, c0, c1)
    acc = lax.dot_general(mean, wl[...], (((1,), (1,)), ((), ())),
                          preferred_element_type=jnp.float32)
    hrelu = jnp.maximum(acc + sf[...] + bl[...], 0.0)
    o_ref[...] = lax.dot_general(hrelu, wlin[...], (((1,), (1,)), ((), ())),
                                 preferred_element_type=jnp.float32) + blin[...]


def _row_spec():
    return pl.BlockSpec((R, D), lambda i: (i, 0))


def _w_spec():
    return pl.BlockSpec((D, D), lambda i: (0, 0))


def _b_spec():
    return pl.BlockSpec((1, D), lambda i: (0, 0))


_tc_self = pl.pallas_call(
    _tc_self_body,
    grid=(N // R,),
    in_specs=[_row_spec(), _w_spec()],
    out_specs=_row_spec(),
    out_shape=jax.ShapeDtypeStruct((N, D), jnp.float32),
)

_tc_combine = pl.pallas_call(
    _tc_combine_body,
    grid=(N // R,),
    in_specs=[_row_spec(), _row_spec(), _row_spec(), _row_spec(),
              _row_spec(), _w_spec(), _b_spec()],
    out_specs=_row_spec(),
    out_shape=jax.ShapeDtypeStruct((N, D), jnp.float32),
)

_tc_final = pl.pallas_call(
    _tc_final_body,
    grid=(N // R,),
    in_specs=[_row_spec(), _row_spec(), _row_spec(), _row_spec(),
              _row_spec(), _w_spec(), _b_spec(), _w_spec(), _b_spec()],
    out_specs=_row_spec(),
    out_shape=jax.ShapeDtypeStruct((N, D), jnp.float32),
)


def kernel(x, edge_index, W1l, b1l, W1r, W2l, b2l, W2r, W3l, b3l, W3r,
           Wlin, blin):
    src = edge_index[0].reshape(NW, EPW)
    dst = edge_index[1].reshape(NW, CH, K)
    b1 = b1l.reshape(1, D)
    b2 = b2l.reshape(1, D)
    b3 = b3l.reshape(1, D)
    bl = blin.reshape(1, D)

    c = _cnt(dst)
    s1 = _tc_self(x, W1r)
    p = _agg(x, src, dst)
    h1 = _tc_combine(p[0], p[1], c[0], c[1], s1, W1l, b1)
    s2 = _tc_self(h1, W2r)
    p = _agg(h1, src, dst)
    h2 = _tc_combine(p[0], p[1], c[0], c[1], s2, W2l, b2)
    s3 = _tc_self(h2, W3r)
    p = _agg(h2, src, dst)
    return _tc_final(p[0], p[1], c[0], c[1], s3, W3l, b3, Wlin, bl)
